# baseline (speedup 1.0000x reference)
import functools

import jax
import jax.numpy as jnp
from jax import lax
from jax.experimental import pallas as pl
from jax.experimental.pallas import tpu as pltpu
from jax.experimental.pallas import tpu_sc as plsc

_VOCAB = 1000000
_EMBED = 64
_CTX = 200
_BATCH = 4096
_SEQ = 200

_NC = 2
_NS = 16
_NW = _NC * _NS
_BPW = _BATCH // _NW
_L = 16


def _emb_kernel(tok_hbm, e_hbm, pb_hbm, out_hbm,
                idxv, gath, blk,
                sem_g0, sem_g1, sem_f0, sem_f1, sem_s0, sem_s1):
    sem_g = (sem_g0, sem_g1)
    sem_f = (sem_f0, sem_f1)
    sem_s = (sem_s0, sem_s1)
    wid = lax.axis_index("s") * _NC + lax.axis_index("c")
    bbase = wid * _BPW
    bvecs = [lax.iota(jnp.int32, _L) + m * _L for m in range(_BPW // _L)]

    def prefetch(s, b):
        pltpu.sync_copy(tok_hbm.at[s, pl.ds(bbase, _BPW)], idxv.at[b])
        pltpu.async_copy(e_hbm.at[idxv.at[b]], gath.at[b], sem_g[b])
        pltpu.async_copy(pb_hbm.at[s], blk.at[b], sem_f[b])

    def wait_gathers(b):
        pltpu.make_async_copy(
            e_hbm.at[pl.ds(0, _BPW)], gath.at[b], sem_g[b]
        ).wait()
        pltpu.make_async_copy(pb_hbm.at[0], blk.at[b], sem_f[b]).wait()

    def wait_store(b):
        pltpu.make_async_copy(
            blk.at[b], out_hbm.at[0, pl.ds(0, _EMBED), pl.ds(0, _BPW)],
            sem_s[b],
        ).wait()

    def transpose_add(b):
        gb = gath.at[b]
        ob = blk.at[b]

        def body(d, c):
            dvec = jnp.full((_L,), 0, jnp.int32) + d
            for m in range(_BPW // _L):
                g = plsc.load_gather(gb, [bvecs[m], dvec])
                plsc.addupdate(ob.at[d, pl.ds(m * _L, _L)], g)
            return c

        lax.fori_loop(0, _EMBED, body, 0)

    prefetch(0, 0)

    def outer(i, carry):
        for b in range(2):
            s = 2 * i + b
            bn = b ^ 1

            @pl.when(s >= 1)
            def _():
                wait_store(bn)

            @pl.when(s + 1 < _SEQ)
            def _():
                prefetch(s + 1, bn)

            wait_gathers(b)
            transpose_add(b)
            pltpu.async_copy(
                blk.at[b],
                out_hbm.at[s, pl.ds(0, _EMBED), pl.ds(bbase, _BPW)],
                sem_s[b],
            )
        return carry

    lax.fori_loop(0, _SEQ // 2, outer, 0)
    wait_store(1)


def kernel(token_batch, E, P):
    tok_t = token_batch.astype(jnp.int32).T
    e_pad = jnp.pad(E, ((0, 0), (0, 2 * _EMBED - _EMBED)))
    p_big = jnp.broadcast_to(
        P[:_SEQ, :, None], (_SEQ, _EMBED, _BPW)
    )
    mesh = plsc.VectorSubcoreMesh(core_axis_name="c", subcore_axis_name="s")
    run = functools.partial(
        pl.kernel,
        mesh=mesh,
        compiler_params=pltpu.CompilerParams(
            use_tc_tiling_on_sc=True, needs_layout_passes=False
        ),
        out_type=jax.ShapeDtypeStruct((_SEQ, _EMBED, _BATCH), jnp.float32),
        scratch_types=[
            pltpu.VMEM((2, _BPW), jnp.int32),
            pltpu.VMEM((2, _BPW, 2 * _EMBED), jnp.float32),
            pltpu.VMEM((2, _EMBED, _BPW), jnp.float32),
            pltpu.SemaphoreType.DMA,
            pltpu.SemaphoreType.DMA,
            pltpu.SemaphoreType.DMA,
            pltpu.SemaphoreType.DMA,
            pltpu.SemaphoreType.DMA,
            pltpu.SemaphoreType.DMA,
        ],
    )(_emb_kernel)
    out_t = run(tok_t, e_pad, p_big)
    return jnp.transpose(out_t, (2, 0, 1))


# speedup vs baseline: 1.0514x; 1.0514x over previous
import functools

import jax
import jax.numpy as jnp
from jax import lax
from jax.experimental import pallas as pl
from jax.experimental.pallas import tpu as pltpu
from jax.experimental.pallas import tpu_sc as plsc

_VOCAB = 1000000
_EMBED = 64
_CTX = 200
_BATCH = 4096
_SEQ = 200

_NC = 2
_NS = 16
_NW = _NC * _NS
_BPW = _BATCH // _NW
_L = 16


def _emb_kernel(tok_hbm, e_hbm, pb_hbm, out_hbm,
                idxv, gath, blk,
                sem_g0, sem_g1, sem_f0, sem_f1, sem_s0, sem_s1):
    sem_g = (sem_g0, sem_g1)
    sem_f = (sem_f0, sem_f1)
    sem_s = (sem_s0, sem_s1)
    wid = lax.axis_index("s") * _NC + lax.axis_index("c")
    bbase = wid * _BPW
    iota = lax.iota(jnp.int32, _L)
    dvecs = [jnp.full((_L,), d, jnp.int32) for d in range(_EMBED)]

    pltpu.sync_copy(
        tok_hbm.at[pl.ds(0, _SEQ), pl.ds(bbase, _BPW)], idxv
    )

    def prefetch(s, b):
        pltpu.async_copy(e_hbm.at[idxv.at[s]], gath.at[b], sem_g[b])
        pltpu.async_copy(pb_hbm.at[s], blk.at[b], sem_f[b])

    def wait_gathers(b):
        pltpu.make_async_copy(
            e_hbm.at[pl.ds(0, _BPW)], gath.at[b], sem_g[b]
        ).wait()
        pltpu.make_async_copy(pb_hbm.at[0], blk.at[b], sem_f[b]).wait()

    def wait_store(b):
        pltpu.make_async_copy(
            blk.at[b], out_hbm.at[0, pl.ds(0, _EMBED), pl.ds(0, _BPW)],
            sem_s[b],
        ).wait()

    def transpose_add(b):
        gb = gath.at[b]
        ob = blk.at[b]

        def body(m, c):
            moff = m * _L
            bvec = iota + moff
            for d in range(_EMBED):
                g = plsc.load_gather(gb, [bvec, dvecs[d]])
                plsc.addupdate(ob.at[d, pl.ds(moff, _L)], g)
            return c

        lax.fori_loop(0, _BPW // _L, body, 0)

    prefetch(0, 0)

    def outer(i, carry):
        for b in range(2):
            s = 2 * i + b
            bn = b ^ 1

            @pl.when(s >= 1)
            def _():
                wait_store(bn)

            @pl.when(s + 1 < _SEQ)
            def _():
                prefetch(s + 1, bn)

            wait_gathers(b)
            transpose_add(b)
            pltpu.async_copy(
                blk.at[b],
                out_hbm.at[s, pl.ds(0, _EMBED), pl.ds(bbase, _BPW)],
                sem_s[b],
            )
        return carry

    lax.fori_loop(0, _SEQ // 2, outer, 0)
    wait_store(1)


def kernel(token_batch, E, P):
    tok_t = token_batch.astype(jnp.int32).T
    e_pad = jnp.pad(E, ((0, 0), (0, 2 * _EMBED - _EMBED)))
    p_big = jnp.broadcast_to(
        P[:_SEQ, :, None], (_SEQ, _EMBED, _BPW)
    )
    mesh = plsc.VectorSubcoreMesh(core_axis_name="c", subcore_axis_name="s")
    run = functools.partial(
        pl.kernel,
        mesh=mesh,
        compiler_params=pltpu.CompilerParams(
            use_tc_tiling_on_sc=True, needs_layout_passes=False
        ),
        out_type=jax.ShapeDtypeStruct((_SEQ, _EMBED, _BATCH), jnp.float32),
        scratch_types=[
            pltpu.VMEM((_SEQ, _BPW), jnp.int32),
            pltpu.VMEM((2, _BPW, 2 * _EMBED), jnp.float32),
            pltpu.VMEM((2, _EMBED, _BPW), jnp.float32),
            pltpu.SemaphoreType.DMA,
            pltpu.SemaphoreType.DMA,
            pltpu.SemaphoreType.DMA,
            pltpu.SemaphoreType.DMA,
            pltpu.SemaphoreType.DMA,
            pltpu.SemaphoreType.DMA,
        ],
    )(_emb_kernel)
    out_t = run(tok_t, e_pad, p_big)
    return jnp.transpose(out_t, (2, 0, 1))


# speedup vs baseline: 1.6068x; 1.5283x over previous
import functools

import jax
import jax.numpy as jnp
from jax import lax
from jax.experimental import pallas as pl
from jax.experimental.pallas import tpu as pltpu
from jax.experimental.pallas import tpu_sc as plsc

_VOCAB = 1000000
_EMBED = 64
_CTX = 200
_BATCH = 4096
_SEQ = 200

_NC = 2
_NS = 16
_NW = _NC * _NS
_BPW = _BATCH // _NW
_L = 16


def _emb_kernel(tok_hbm, e_hbm, pb_hbm, out_hbm,
                idxv, gath, blk,
                sem_g0, sem_g1, sem_f0, sem_f1, sem_s0, sem_s1):
    sem_g = (sem_g0, sem_g1)
    sem_f = (sem_f0, sem_f1)
    sem_s = (sem_s0, sem_s1)
    wid = lax.axis_index("s") * _NC + lax.axis_index("c")
    bbase = wid * _BPW
    iota = lax.iota(jnp.int32, _L)
    mods = [(iota + k) & (_L - 1) for k in range(_L)]

    pltpu.sync_copy(
        tok_hbm.at[pl.ds(0, _SEQ), pl.ds(bbase, _BPW)], idxv
    )

    def prefetch(s, b):
        pltpu.async_copy(e_hbm.at[idxv.at[s]], gath.at[b], sem_g[b])
        pltpu.async_copy(pb_hbm.at[s], blk.at[b], sem_f[b])

    def wait_gathers(b):
        pltpu.make_async_copy(
            e_hbm.at[pl.ds(0, _BPW)], gath.at[b], sem_g[b]
        ).wait()
        pltpu.make_async_copy(pb_hbm.at[0], blk.at[b], sem_f[b]).wait()

    def wait_store(b):
        pltpu.make_async_copy(
            blk.at[b], out_hbm.at[0, pl.ds(0, _EMBED), pl.ds(0, _BPW)],
            sem_s[b],
        ).wait()

    def transpose_add(b):
        gb = gath.at[b]
        ob = blk.at[b]

        def body(m, c):
            bvec = iota + m * _L
            for d0 in range(0, _EMBED, _L):
                for k in range(_L):
                    dvec = mods[k] + d0
                    g = plsc.load_gather(gb, [bvec, dvec])
                    plsc.addupdate_scatter(ob, [dvec, bvec], g)
            return c

        lax.fori_loop(0, _BPW // _L, body, 0)

    prefetch(0, 0)

    def outer(i, carry):
        for b in range(2):
            s = 2 * i + b
            bn = b ^ 1

            @pl.when(s >= 1)
            def _():
                wait_store(bn)

            @pl.when(s + 1 < _SEQ)
            def _():
                prefetch(s + 1, bn)

            wait_gathers(b)
            transpose_add(b)
            pltpu.async_copy(
                blk.at[b],
                out_hbm.at[s, pl.ds(0, _EMBED), pl.ds(bbase, _BPW)],
                sem_s[b],
            )
        return carry

    lax.fori_loop(0, _SEQ // 2, outer, 0)
    wait_store(1)


def kernel(token_batch, E, P):
    tok_t = token_batch.astype(jnp.int32).T
    e_pad = jnp.pad(E, ((0, 0), (0, 2 * _EMBED - _EMBED)))
    p_big = jnp.broadcast_to(
        P[:_SEQ, :, None], (_SEQ, _EMBED, _BPW)
    )
    mesh = plsc.VectorSubcoreMesh(core_axis_name="c", subcore_axis_name="s")
    run = functools.partial(
        pl.kernel,
        mesh=mesh,
        compiler_params=pltpu.CompilerParams(
            use_tc_tiling_on_sc=True, needs_layout_passes=False
        ),
        out_type=jax.ShapeDtypeStruct((_SEQ, _EMBED, _BATCH), jnp.float32),
        scratch_types=[
            pltpu.VMEM((_SEQ, _BPW), jnp.int32),
            pltpu.VMEM((2, _BPW, 2 * _EMBED), jnp.float32),
            pltpu.VMEM((2, _EMBED, _BPW), jnp.float32),
            pltpu.SemaphoreType.DMA,
            pltpu.SemaphoreType.DMA,
            pltpu.SemaphoreType.DMA,
            pltpu.SemaphoreType.DMA,
            pltpu.SemaphoreType.DMA,
            pltpu.SemaphoreType.DMA,
        ],
    )(_emb_kernel)
    out_t = run(tok_t, e_pad, p_big)
    return jnp.transpose(out_t, (2, 0, 1))


# speedup vs baseline: 1.6556x; 1.0304x over previous
import functools

import jax
import jax.numpy as jnp
from jax import lax
from jax.experimental import pallas as pl
from jax.experimental.pallas import tpu as pltpu
from jax.experimental.pallas import tpu_sc as plsc

_VOCAB = 1000000
_EMBED = 64
_CTX = 200
_BATCH = 4096
_SEQ = 200

_NC = 2
_NS = 16
_NW = _NC * _NS
_BPW = _BATCH // _NW
_L = 16


def _emb_kernel(tok_hbm, e_hbm, p_hbm, out_hbm,
                idxv, gath, blk, p_v,
                sem_g0, sem_g1, sem_s0, sem_s1):
    sem_g = (sem_g0, sem_g1)
    sem_s = (sem_s0, sem_s1)
    wid = lax.axis_index("s") * _NC + lax.axis_index("c")
    bbase = wid * _BPW
    iota = lax.iota(jnp.int32, _L)
    mods = [(iota + k) & (_L - 1) for k in range(_L)]

    pltpu.sync_copy(tok_hbm.at[pl.ds(0, _SEQ), pl.ds(bbase, _BPW)], idxv)
    pltpu.sync_copy(p_hbm, p_v)

    def prefetch(s, b):
        pltpu.async_copy(e_hbm.at[idxv.at[s]], gath.at[b], sem_g[b])

    def wait_gathers(b):
        pltpu.make_async_copy(
            e_hbm.at[pl.ds(0, _BPW)], gath.at[b], sem_g[b]
        ).wait()

    def wait_store(b):
        pltpu.make_async_copy(
            blk.at[b], out_hbm.at[0, pl.ds(0, _EMBED), pl.ds(0, _BPW)],
            sem_s[b],
        ).wait()

    def transpose_add(s, b):
        gb = gath.at[b]
        ob = blk.at[b]
        pvecs = [p_v[s, pl.ds(d0, _L)] for d0 in range(0, _EMBED, _L)]

        def body(m, c):
            moff = m * _L
            for t, d0 in enumerate(range(0, _EMBED, _L)):
                dvec = iota + d0
                pvec = pvecs[t]
                for k in range(_L):
                    bvec = mods[k] + moff
                    g = plsc.load_gather(gb, [bvec, dvec])
                    plsc.store_scatter(ob, [dvec, bvec], g + pvec)
            return c

        lax.fori_loop(0, _BPW // _L, body, 0)

    prefetch(0, 0)

    def outer(i, carry):
        for b in range(2):
            s = 2 * i + b
            bn = b ^ 1

            @pl.when(s >= 1)
            def _():
                wait_store(bn)

            @pl.when(s + 1 < _SEQ)
            def _():
                prefetch(s + 1, bn)

            wait_gathers(b)
            transpose_add(s, b)
            pltpu.async_copy(
                blk.at[b],
                out_hbm.at[s, pl.ds(0, _EMBED), pl.ds(bbase, _BPW)],
                sem_s[b],
            )
        return carry

    lax.fori_loop(0, _SEQ // 2, outer, 0)
    wait_store(1)


def kernel(token_batch, E, P):
    tok_t = token_batch.astype(jnp.int32).T
    e_pad = jnp.pad(E, ((0, 0), (0, 2 * _EMBED - _EMBED)))
    mesh = plsc.VectorSubcoreMesh(core_axis_name="c", subcore_axis_name="s")
    run = functools.partial(
        pl.kernel,
        mesh=mesh,
        compiler_params=pltpu.CompilerParams(
            use_tc_tiling_on_sc=True, needs_layout_passes=False
        ),
        out_type=jax.ShapeDtypeStruct((_SEQ, _EMBED, _BATCH), jnp.float32),
        scratch_types=[
            pltpu.VMEM((_SEQ, _BPW), jnp.int32),
            pltpu.VMEM((2, _BPW, 2 * _EMBED), jnp.float32),
            pltpu.VMEM((2, _EMBED, _BPW), jnp.float32),
            pltpu.VMEM((_CTX, _EMBED), jnp.float32),
            pltpu.SemaphoreType.DMA,
            pltpu.SemaphoreType.DMA,
            pltpu.SemaphoreType.DMA,
            pltpu.SemaphoreType.DMA,
        ],
    )(_emb_kernel)
    out_t = run(tok_t, e_pad, P)
    return jnp.transpose(out_t, (2, 0, 1))


# speedup vs baseline: 1.7825x; 1.0767x over previous
import functools

import jax
import jax.numpy as jnp
from jax import lax
from jax.experimental import pallas as pl
from jax.experimental.pallas import tpu as pltpu
from jax.experimental.pallas import tpu_sc as plsc

_VOCAB = 1000000
_EMBED = 64
_CTX = 200
_BATCH = 4096
_SEQ = 200

_NC = 2
_NS = 16
_NW = _NC * _NS
_BPW = _BATCH // _NW
_L = 16


def _emb_kernel(tok_hbm, e_hbm, p_hbm, out_hbm,
                idxv, gath, blk, p_v,
                sem_g0, sem_g1, sem_g2, sem_g3, sem_s0, sem_s1):
    sem_g = (sem_g0, sem_g1, sem_g2, sem_g3)
    sem_s = (sem_s0, sem_s1)
    wid = lax.axis_index("s") * _NC + lax.axis_index("c")
    bbase = wid * _BPW
    iota = lax.iota(jnp.int32, _L)
    mods = [(iota + k) & (_L - 1) for k in range(_L)]

    pltpu.sync_copy(tok_hbm.at[pl.ds(0, _SEQ), pl.ds(bbase, _BPW)], idxv)
    pltpu.sync_copy(p_hbm, p_v)

    def prefetch(s, b):
        pltpu.async_copy(e_hbm.at[idxv.at[s]], gath.at[b], sem_g[b])

    def wait_gathers(b):
        pltpu.make_async_copy(
            e_hbm.at[pl.ds(0, _BPW)], gath.at[b], sem_g[b]
        ).wait()

    def wait_store(b):
        pltpu.make_async_copy(
            blk.at[b], out_hbm.at[0, pl.ds(0, _EMBED), pl.ds(0, _BPW)],
            sem_s[b],
        ).wait()

    def transpose_add(s, gb_i, ob_i):
        gb = gath.at[gb_i]
        ob = blk.at[ob_i]
        prow = s >> 1
        pcol = (s & 1) * _EMBED
        pvecs = [p_v[prow, pl.ds(pcol + d0, _L)]
                 for d0 in range(0, _EMBED, _L)]

        def body(m, c):
            moff = m * _L
            for t, d0 in enumerate(range(0, _EMBED, _L)):
                dvec = iota + d0
                pvec = pvecs[t]
                for k in range(_L):
                    bvec = mods[k] + moff
                    g = plsc.load_gather(gb, [bvec, dvec])
                    plsc.store_scatter(ob, [dvec, bvec], g + pvec)
            return c

        lax.fori_loop(0, _BPW // _L, body, 0)

    prefetch(0, 0)
    prefetch(1, 1)

    def outer(i, carry):
        for b in range(4):
            s = 4 * i + b
            ob_i = b & 1

            @pl.when(s >= 2)
            def _():
                wait_store(ob_i)

            @pl.when(s + 2 < _SEQ)
            def _():
                prefetch(s + 2, (b + 2) & 3)

            wait_gathers(b)
            transpose_add(s, b, ob_i)
            pltpu.async_copy(
                blk.at[ob_i],
                out_hbm.at[s, pl.ds(0, _EMBED), pl.ds(bbase, _BPW)],
                sem_s[ob_i],
            )
        return carry

    lax.fori_loop(0, _SEQ // 4, outer, 0)
    wait_store(0)
    wait_store(1)


def kernel(token_batch, E, P):
    tok_t = token_batch.astype(jnp.int32).T
    e_pad = jnp.pad(E, ((0, 0), (0, 2 * _EMBED - _EMBED)))
    mesh = plsc.VectorSubcoreMesh(core_axis_name="c", subcore_axis_name="s")
    run = functools.partial(
        pl.kernel,
        mesh=mesh,
        compiler_params=pltpu.CompilerParams(
            use_tc_tiling_on_sc=True, needs_layout_passes=False
        ),
        out_type=jax.ShapeDtypeStruct((_SEQ, _EMBED, _BATCH), jnp.float32),
        scratch_types=[
            pltpu.VMEM((_SEQ, _BPW), jnp.int32),
            pltpu.VMEM((4, _BPW, 2 * _EMBED), jnp.float32),
            pltpu.VMEM((2, _EMBED, _BPW), jnp.float32),
            pltpu.VMEM((_CTX // 2, 2 * _EMBED), jnp.float32),
            pltpu.SemaphoreType.DMA,
            pltpu.SemaphoreType.DMA,
            pltpu.SemaphoreType.DMA,
            pltpu.SemaphoreType.DMA,
            pltpu.SemaphoreType.DMA,
            pltpu.SemaphoreType.DMA,
        ],
    )(_emb_kernel)
    out_t = run(tok_t, e_pad, P.reshape(_CTX // 2, 2 * _EMBED))
    return jnp.transpose(out_t, (2, 0, 1))
